# baseline (speedup 1.0000x reference)
import functools

import jax
import jax.numpy as jnp
from jax import lax
from jax.experimental import pallas as pl
from jax.experimental.pallas import tpu as pltpu
from jax.experimental.pallas import tpu_sc as plsc

_BOS_IDX = 1000001
_EOS_IDX = 1000002
_EMBED = 64
_NC = 2
_NS = 16
_NW = _NC * _NS
_CHUNK = 128
_K = 6
_NBUF = 2


@functools.partial(jax.jit, static_argnums=(2, 3))
def _sc_gather(table, idx_blocks, per_w, n_chunks):
    n_rows = _NW * per_w
    group_rows = _K * _CHUNK
    n_groups = per_w // group_rows
    tail_chunks = n_chunks - n_groups * _K
    tail_rows = per_w - n_groups * group_rows
    mesh = plsc.VectorSubcoreMesh(core_axis_name="c", subcore_axis_name="s")

    @functools.partial(
        pl.kernel,
        mesh=mesh,
        out_type=jax.ShapeDtypeStruct((n_rows, _EMBED), jnp.bfloat16),
        scratch_types=[
            pltpu.VMEM((n_chunks, _CHUNK), jnp.int32),
            pltpu.VMEM((_NBUF, group_rows, _EMBED), jnp.bfloat16),
            pltpu.SemaphoreType.DMA,
            pltpu.SemaphoreType.DMA,
        ],
        compiler_params=pltpu.CompilerParams(use_tc_tiling_on_sc=False),
    )
    def k(table_hbm, idx_hbm, out_hbm, idx_v, rows_v, sem0, sem1):
        wid = lax.axis_index("s") * _NC + lax.axis_index("c")
        base = wid * per_w
        sems = (sem0, sem1)
        pltpu.sync_copy(idx_hbm.at[wid], idx_v)

        def fire(first_chunk, count, b):
            cps = []
            for j in range(count):
                cps.append(pltpu.async_copy(
                    table_hbm.at[idx_v.at[first_chunk + j]],
                    rows_v.at[b, pl.ds(j * _CHUNK, _CHUNK)],
                    sems[b],
                ))
            return cps

        pending = [fire(0, _K, 0), fire(_K, _K, 1)]
        for g in range(n_groups):
            b = g % _NBUF
            for cp in pending[b]:
                cp.wait()
            pltpu.sync_copy(
                rows_v.at[b],
                out_hbm.at[pl.ds(base + g * group_rows, group_rows)],
            )
            ng = g + _NBUF
            if ng < n_groups:
                pending[b] = fire(ng * _K, _K, b)
            elif ng == n_groups and tail_chunks:
                pending[b] = fire(ng * _K, tail_chunks, b)
            else:
                pending[b] = []
        if tail_chunks:
            b = n_groups % _NBUF
            for cp in pending[b]:
                cp.wait()
            pltpu.sync_copy(
                rows_v.at[b, pl.ds(0, tail_rows)],
                out_hbm.at[pl.ds(base + n_groups * group_rows, tail_rows)],
            )

    return k(table, idx_blocks)


def kernel(indices, table):
    B, L = indices.shape
    bos = jnp.full((B, 1), _BOS_IDX, jnp.int32)
    eos = jnp.full((B, 1), _EOS_IDX, jnp.int32)
    idx = jnp.concatenate([bos, indices.astype(jnp.int32), eos], axis=1)
    n_rows = B * (L + 2)
    per_w = n_rows // _NW
    n_chunks = -(-per_w // _CHUNK)
    pad = n_chunks * _CHUNK - per_w
    idx_blocks = jnp.pad(idx.reshape(_NW, per_w), ((0, 0), (0, pad)))
    idx_blocks = idx_blocks.reshape(_NW, n_chunks, _CHUNK)
    out = _sc_gather(table.astype(jnp.bfloat16), idx_blocks, per_w, n_chunks)
    return out.astype(jnp.float32).reshape(B, L + 2, _EMBED)


# speedup vs baseline: 1.7439x; 1.7439x over previous
import functools

import jax
import jax.numpy as jnp
from jax import lax
from jax.experimental import pallas as pl
from jax.experimental.pallas import tpu as pltpu
from jax.experimental.pallas import tpu_sc as plsc

_BOS_IDX = 1000001
_EOS_IDX = 1000002
_EMBED = 64
_NC = 2
_NS = 16
_NW = _NC * _NS
_CHUNK = 128
_K = 6
_NBUF = 2


@functools.partial(jax.jit, static_argnums=(2, 3))
def _sc_gather(table, idx_blocks, per_w, n_chunks):
    n_rows = _NW * per_w
    group_rows = _K * _CHUNK
    n_groups = per_w // group_rows
    tail_chunks = n_chunks - n_groups * _K
    tail_rows = per_w - n_groups * group_rows
    mesh = plsc.VectorSubcoreMesh(core_axis_name="c", subcore_axis_name="s")

    @functools.partial(
        pl.kernel,
        mesh=mesh,
        out_type=jax.ShapeDtypeStruct((n_rows, _EMBED), jnp.float32),
        scratch_types=[
            pltpu.VMEM((n_chunks, _CHUNK), jnp.int32),
            pltpu.VMEM((_NBUF, group_rows, _EMBED), jnp.float32),
            pltpu.SemaphoreType.DMA,
            pltpu.SemaphoreType.DMA,
        ],
        compiler_params=pltpu.CompilerParams(use_tc_tiling_on_sc=False),
    )
    def k(table_hbm, idx_hbm, out_hbm, idx_v, rows_v, sem0, sem1):
        wid = lax.axis_index("s") * _NC + lax.axis_index("c")
        base = wid * per_w
        sems = (sem0, sem1)
        pltpu.sync_copy(idx_hbm.at[wid], idx_v)

        def fire(first_chunk, count, b):
            cps = []
            for j in range(count):
                cps.append(pltpu.async_copy(
                    table_hbm.at[idx_v.at[first_chunk + j]],
                    rows_v.at[b, pl.ds(j * _CHUNK, _CHUNK)],
                    sems[b],
                ))
            return cps

        pending = [fire(0, _K, 0), fire(_K, _K, 1)]
        for g in range(n_groups):
            b = g % _NBUF
            for cp in pending[b]:
                cp.wait()
            pltpu.sync_copy(
                rows_v.at[b],
                out_hbm.at[pl.ds(base + g * group_rows, group_rows)],
            )
            ng = g + _NBUF
            if ng < n_groups:
                pending[b] = fire(ng * _K, _K, b)
            elif ng == n_groups and tail_chunks:
                pending[b] = fire(ng * _K, tail_chunks, b)
            else:
                pending[b] = []
        if tail_chunks:
            b = n_groups % _NBUF
            for cp in pending[b]:
                cp.wait()
            pltpu.sync_copy(
                rows_v.at[b, pl.ds(0, tail_rows)],
                out_hbm.at[pl.ds(base + n_groups * group_rows, tail_rows)],
            )

    return k(table, idx_blocks)


def kernel(indices, table):
    B, L = indices.shape
    bos = jnp.full((B, 1), _BOS_IDX, jnp.int32)
    eos = jnp.full((B, 1), _EOS_IDX, jnp.int32)
    idx = jnp.concatenate([bos, indices.astype(jnp.int32), eos], axis=1)
    n_rows = B * (L + 2)
    per_w = n_rows // _NW
    n_chunks = -(-per_w // _CHUNK)
    pad = n_chunks * _CHUNK - per_w
    idx_blocks = jnp.pad(idx.reshape(_NW, per_w), ((0, 0), (0, pad)))
    idx_blocks = idx_blocks.reshape(_NW, n_chunks, _CHUNK)
    out = _sc_gather(table, idx_blocks, per_w, n_chunks)
    return out.reshape(B, L + 2, _EMBED)
